# baseline (speedup 1.0000x reference)
import functools

import jax
import jax.numpy as jnp
from jax import lax
from jax.experimental import pallas as pl
from jax.experimental.pallas import tpu as pltpu
from jax.experimental.pallas import tpu_sc as plsc

B = 128
K = 36
L = 20
WORD_DIM = 300
QUES_DIM = 1024
V_DIM = 2048
SCORE_DIM = 1024
NEG_SLOPE = 0.01
WORD_PAD = 320

BB = 8


def _lrelu(x):
    return jnp.where(x >= 0, x, NEG_SLOPE * x)


def _sc_gather(table, idx):
    info = plsc.get_sparse_core_info()
    nc, ns = info.num_cores, info.num_subcores
    nw = nc * ns
    n = idx.shape[0]
    d = table.shape[1]
    b_per_w = n // nw
    mesh = plsc.VectorSubcoreMesh(core_axis_name="c", subcore_axis_name="s")

    @functools.partial(
        pl.kernel,
        mesh=mesh,
        out_type=jax.ShapeDtypeStruct((n, d), jnp.float32),
        scratch_types=[
            pltpu.VMEM((b_per_w,), jnp.int32),
            pltpu.VMEM((b_per_w, d), jnp.float32),
            pltpu.SemaphoreType.DMA,
        ],
        compiler_params=pltpu.CompilerParams(use_tc_tiling_on_sc=False),
    )
    def gather_k(table_hbm, idx_hbm, out_hbm, idx_v, rows_v, sem):
        wid = lax.axis_index("s") * nc + lax.axis_index("c")
        base = wid * b_per_w
        pltpu.sync_copy(idx_hbm.at[pl.ds(base, b_per_w)], idx_v)
        pltpu.async_copy(table_hbm.at[idx_v], rows_v, sem).wait()
        pltpu.sync_copy(rows_v, out_hbm.at[pl.ds(base, b_per_w)])

    return gather_k(table, idx)


def _gru_body(xe_ref, w_ih_ref, w_hh_ref, b_ih_ref, b_hh_ref, qp_w_ref,
              qp_b_ref, ws_w_ref, rw_w_ref, rw_b_ref, t_ref, wq_ref, h_ref):
    l = pl.program_id(0)

    @pl.when(l == 0)
    def _():
        h_ref[...] = jnp.zeros((B, QUES_DIM), jnp.float32)

    x = xe_ref[0]
    h = h_ref[...]
    dn = (((1,), (1,)), ((), ()))
    gi = lax.dot_general(x, w_ih_ref[...], dn,
                         preferred_element_type=jnp.float32, precision=lax.Precision.HIGHEST) + b_ih_ref[...]
    gh = lax.dot_general(h, w_hh_ref[...], dn,
                         preferred_element_type=jnp.float32, precision=lax.Precision.HIGHEST) + b_hh_ref[...]
    i_r = gi[:, :QUES_DIM]
    i_z = gi[:, QUES_DIM:2 * QUES_DIM]
    i_n = gi[:, 2 * QUES_DIM:]
    h_r = gh[:, :QUES_DIM]
    h_z = gh[:, QUES_DIM:2 * QUES_DIM]
    h_n = gh[:, 2 * QUES_DIM:]
    r = jax.nn.sigmoid(i_r + h_r)
    z = jax.nn.sigmoid(i_z + h_z)
    n = jnp.tanh(i_n + r * h_n)
    h_new = (1.0 - z) * n + z * h
    h_ref[...] = h_new

    @pl.when(l == L - 1)
    def _():
        q_p = _lrelu(lax.dot_general(h_new, qp_w_ref[...], dn,
                                     preferred_element_type=jnp.float32, precision=lax.Precision.HIGHEST)
                     + qp_b_ref[...])
        t_ref[...] = q_p * ws_w_ref[...]
        wq_ref[...] = jnp.sum(h_new * rw_w_ref[...], axis=1,
                              keepdims=True) + rw_b_ref[...]


def _gru(xe, w_ih, w_hh, b_ih, b_hh, qp_w, qp_b, ws_w, rw_w, rw_b):
    full = lambda s: pl.BlockSpec(s, lambda l: (0,) * len(s))
    return pl.pallas_call(
        _gru_body,
        grid=(L,),
        in_specs=[
            pl.BlockSpec((1, B, WORD_PAD), lambda l: (l, 0, 0)),
            full((3 * QUES_DIM, WORD_PAD)),
            full((3 * QUES_DIM, QUES_DIM)),
            full((1, 3 * QUES_DIM)),
            full((1, 3 * QUES_DIM)),
            full((SCORE_DIM, QUES_DIM)),
            full((1, SCORE_DIM)),
            full((1, SCORE_DIM)),
            full((1, QUES_DIM)),
            full((1, 1)),
        ],
        out_specs=[
            pl.BlockSpec((B, SCORE_DIM), lambda l: (0, 0)),
            pl.BlockSpec((B, 1), lambda l: (0, 0)),
        ],
        out_shape=[
            jax.ShapeDtypeStruct((B, SCORE_DIM), jnp.float32),
            jax.ShapeDtypeStruct((B, 1), jnp.float32),
        ],
        scratch_shapes=[pltpu.VMEM((B, QUES_DIM), jnp.float32)],
    )(xe, w_ih, w_hh, b_ih.reshape(1, -1), b_hh.reshape(1, -1), qp_w,
      qp_b.reshape(1, -1), ws_w, rw_w, rw_b.reshape(1, -1))


def _score_body(v_ref, b_ref, t_ref, wq_ref, vp_w_ref, vp_b_ref, fr_wt_ref,
                fr_b_ref, d_w_ref, sc_ref, out_ref):
    dn = (((1,), (1,)), ((), ()))
    v2 = v_ref[...].reshape(BB * K, V_DIM)
    vp = _lrelu(lax.dot_general(v2, vp_w_ref[...], dn,
                                preferred_element_type=jnp.float32, precision=lax.Precision.HIGHEST)
                + vp_b_ref[...])
    vp3 = vp.reshape(BB, K, SCORE_DIM)
    ws_b = sc_ref[0, 0]
    d_b = sc_ref[0, 1]
    kappa = jnp.sum(vp3 * t_ref[...][:, None, :], axis=2) + ws_b

    ss = jnp.sum(v2 * v2, axis=1)
    inv = 1.0 / jnp.maximum(jnp.sqrt(ss), 1e-12)
    nv2 = v2 * inv[:, None]
    big = lax.dot_general(nv2, nv2, dn,
                          preferred_element_type=jnp.float32, precision=lax.Precision.HIGHEST)
    vtv = jnp.concatenate(
        [big[i * K:(i + 1) * K, i * K:(i + 1) * K][None] for i in range(BB)],
        axis=0)

    bx = b_ref[...]
    x0 = bx[:, :, 0]
    y0 = bx[:, :, 1]
    x1 = bx[:, :, 2]
    y1 = bx[:, :, 3]
    area = (x1 - x0) * (y1 - y0)
    lr = jnp.minimum(x1[:, None, :], x1[:, :, None]) - \
        jnp.maximum(x0[:, None, :], x0[:, :, None])
    ud = jnp.minimum(y1[:, None, :], y1[:, :, None]) - \
        jnp.maximum(y0[:, None, :], y0[:, :, None])
    overlap = jnp.maximum(lr, 0.0) * jnp.maximum(ud, 0.0)
    a_j = area[:, None, :]
    a_i = area[:, :, None]
    iou = overlap / (a_j + a_i - overlap)
    o_ij = overlap / a_j
    o_ji = overlap / a_i

    fr_wt = fr_wt_ref[...]
    b2 = bx.reshape(BB * K, 6)
    dnn = (((1,), (0,)), ((), ()))
    u = lax.dot_general(b2, fr_wt[2:8], dnn,
                        preferred_element_type=jnp.float32, precision=lax.Precision.HIGHEST)
    w2m = lax.dot_general(b2, fr_wt[8:14], dnn,
                          preferred_element_type=jnp.float32, precision=lax.Precision.HIGHEST)
    u3 = u.reshape(BB, K, 100)
    w23 = w2m.reshape(BB, K, 100)

    f = lambda c: fr_wt[c].reshape(1, 1, 1, 100)
    hf = (wq_ref[...].reshape(BB, 1, 1, 1) * f(0)
          + vtv[..., None] * f(1)
          + u3[:, None, :, :]
          + w23[:, :, None, :]
          + iou[..., None] * f(14)
          + o_ij[..., None] * f(15)
          + o_ji[..., None] * f(16)
          + fr_b_ref[...].reshape(1, 1, 1, 100))
    rho = jnp.sum(_lrelu(hf) * d_w_ref[...].reshape(1, 1, 1, 100),
                  axis=3) + d_b
    out_ref[...] = jnp.concatenate([kappa[:, :, None], rho], axis=2)


def _score(v_emb, bboxes, t, wq, vp_w, vp_b, fr_w, fr_b, d_w, scalars):
    nblk = B // BB
    full = lambda s: pl.BlockSpec(s, lambda g: (0,) * len(s))
    return pl.pallas_call(
        _score_body,
        grid=(nblk,),
        in_specs=[
            pl.BlockSpec((BB, K, V_DIM), lambda g: (g, 0, 0)),
            pl.BlockSpec((BB, K, 6), lambda g: (g, 0, 0)),
            pl.BlockSpec((BB, SCORE_DIM), lambda g: (g, 0)),
            pl.BlockSpec((BB, 1), lambda g: (g, 0)),
            full((SCORE_DIM, V_DIM)),
            full((1, SCORE_DIM)),
            full((17, 100)),
            full((1, 100)),
            full((1, 100)),
            full((1, 2)),
        ],
        out_specs=pl.BlockSpec((BB, K, K + 1), lambda g: (g, 0, 0)),
        out_shape=jax.ShapeDtypeStruct((B, K, K + 1), jnp.float32),
    )(v_emb, bboxes, t, wq, vp_w, vp_b.reshape(1, -1), fr_w.T,
      fr_b.reshape(1, -1), d_w, scalars)


def kernel(v_emb, b, q, embd, w_ih, w_hh, b_ih, b_hh, vp_w, vp_b, qp_w,
           qp_b, ws_w, ws_b, rw_w, rw_b, fr_w, fr_b, d_w, d_b):
    idx = q.T.reshape(-1).astype(jnp.int32)
    pad = WORD_PAD - WORD_DIM
    embd_p = jnp.pad(embd, ((0, 0), (0, pad)))
    w_ih_p = jnp.pad(w_ih, ((0, 0), (0, pad)))
    xe_flat = _sc_gather(embd_p, idx)
    xe = xe_flat.reshape(L, B, WORD_PAD)
    t, wq = _gru(xe, w_ih_p, w_hh, b_ih, b_hh, qp_w, qp_b, ws_w, rw_w, rw_b)
    scalars = jnp.stack([ws_b[0], d_b[0]]).reshape(1, 2)
    return _score(v_emb, b, t, wq, vp_w, vp_b, fr_w, fr_b, d_w, scalars)


# speedup vs baseline: 1.9193x; 1.9193x over previous
import functools

import jax
import jax.numpy as jnp
from jax import lax
from jax.experimental import pallas as pl
from jax.experimental.pallas import tpu as pltpu
from jax.experimental.pallas import tpu_sc as plsc

B = 128
K = 36
L = 20
WORD_DIM = 300
QUES_DIM = 1024
V_DIM = 2048
SCORE_DIM = 1024
NEG_SLOPE = 0.01
WORD_PAD = 320

BB = 8


def _lrelu(x):
    return jnp.where(x >= 0, x, NEG_SLOPE * x)


def _sc_gather(table, idx):
    info = plsc.get_sparse_core_info()
    nc, ns = info.num_cores, info.num_subcores
    nw = nc * ns
    n = idx.shape[0]
    d = table.shape[1]
    b_per_w = n // nw
    mesh = plsc.VectorSubcoreMesh(core_axis_name="c", subcore_axis_name="s")

    @functools.partial(
        pl.kernel,
        mesh=mesh,
        out_type=jax.ShapeDtypeStruct((n, d), jnp.float32),
        scratch_types=[
            pltpu.VMEM((b_per_w,), jnp.int32),
            pltpu.VMEM((b_per_w, d), jnp.float32),
            pltpu.SemaphoreType.DMA,
        ],
        compiler_params=pltpu.CompilerParams(use_tc_tiling_on_sc=False),
    )
    def gather_k(table_hbm, idx_hbm, out_hbm, idx_v, rows_v, sem):
        wid = lax.axis_index("s") * nc + lax.axis_index("c")
        base = wid * b_per_w
        pltpu.sync_copy(idx_hbm.at[pl.ds(base, b_per_w)], idx_v)
        pltpu.async_copy(table_hbm.at[idx_v], rows_v, sem).wait()
        pltpu.sync_copy(rows_v, out_hbm.at[pl.ds(base, b_per_w)])

    return gather_k(table, idx)


def _pad_body(in_ref, out_ref):
    rows = in_ref.shape[0]
    out_ref[...] = jnp.concatenate(
        [in_ref[...], jnp.zeros((rows, WORD_PAD - WORD_DIM), jnp.float32)],
        axis=1)


def _pad_table(table):
    rows = table.shape[0]
    blk = 2688
    nblk = pl.cdiv(rows, blk)
    return pl.pallas_call(
        _pad_body,
        grid=(nblk,),
        in_specs=[pl.BlockSpec((blk, WORD_DIM), lambda g: (g, 0))],
        out_specs=pl.BlockSpec((blk, WORD_PAD), lambda g: (g, 0)),
        out_shape=jax.ShapeDtypeStruct((rows, WORD_PAD), jnp.float32),
    )(table)


def _gi_body(xe_ref, w_ih_ref, b_ih_ref, gi_ref):
    x = xe_ref[...][:, :WORD_DIM]
    gi_ref[...] = lax.dot_general(x, w_ih_ref[...], (((1,), (1,)), ((), ())),
                                  preferred_element_type=jnp.float32) \
        + b_ih_ref[...]


def _gi_all(xe_flat, w_ih, b_ih):
    mb = 640
    nblk = (L * B) // mb
    return pl.pallas_call(
        _gi_body,
        grid=(nblk,),
        in_specs=[
            pl.BlockSpec((mb, WORD_PAD), lambda g: (g, 0)),
            pl.BlockSpec((3 * QUES_DIM, WORD_DIM), lambda g: (0, 0)),
            pl.BlockSpec((1, 3 * QUES_DIM), lambda g: (0, 0)),
        ],
        out_specs=pl.BlockSpec((mb, 3 * QUES_DIM), lambda g: (g, 0)),
        out_shape=jax.ShapeDtypeStruct((L * B, 3 * QUES_DIM), jnp.float32),
    )(xe_flat, w_ih, b_ih.reshape(1, -1))


def _gru_body(gi_ref, w_hh_ref, b_hh_ref, qp_w_ref,
              qp_b_ref, ws_w_ref, rw_w_ref, rw_b_ref, t_ref, wq_ref, h_ref):
    l = pl.program_id(0)

    @pl.when(l == 0)
    def _():
        h_ref[...] = jnp.zeros((B, QUES_DIM), jnp.float32)

    h = h_ref[...]
    dn = (((1,), (1,)), ((), ()))
    gi = gi_ref[0]
    gh = lax.dot_general(h, w_hh_ref[...], dn,
                         preferred_element_type=jnp.float32) + b_hh_ref[...]
    i_r = gi[:, :QUES_DIM]
    i_z = gi[:, QUES_DIM:2 * QUES_DIM]
    i_n = gi[:, 2 * QUES_DIM:]
    h_r = gh[:, :QUES_DIM]
    h_z = gh[:, QUES_DIM:2 * QUES_DIM]
    h_n = gh[:, 2 * QUES_DIM:]
    r = jax.nn.sigmoid(i_r + h_r)
    z = jax.nn.sigmoid(i_z + h_z)
    n = jnp.tanh(i_n + r * h_n)
    h_new = (1.0 - z) * n + z * h
    h_ref[...] = h_new

    @pl.when(l == L - 1)
    def _():
        q_p = _lrelu(lax.dot_general(h_new, qp_w_ref[...], dn,
                                     preferred_element_type=jnp.float32)
                     + qp_b_ref[...])
        t_ref[...] = q_p * ws_w_ref[...]
        wq_ref[...] = jnp.sum(h_new * rw_w_ref[...], axis=1,
                              keepdims=True) + rw_b_ref[...]


def _gru(gi_all, w_hh, b_hh, qp_w, qp_b, ws_w, rw_w, rw_b):
    full = lambda s: pl.BlockSpec(s, lambda l: (0,) * len(s))
    return pl.pallas_call(
        _gru_body,
        grid=(L,),
        in_specs=[
            pl.BlockSpec((1, B, 3 * QUES_DIM), lambda l: (l, 0, 0)),
            full((3 * QUES_DIM, QUES_DIM)),
            full((1, 3 * QUES_DIM)),
            full((SCORE_DIM, QUES_DIM)),
            full((1, SCORE_DIM)),
            full((1, SCORE_DIM)),
            full((1, QUES_DIM)),
            full((1, 1)),
        ],
        out_specs=[
            pl.BlockSpec((B, SCORE_DIM), lambda l: (0, 0)),
            pl.BlockSpec((B, 1), lambda l: (0, 0)),
        ],
        out_shape=[
            jax.ShapeDtypeStruct((B, SCORE_DIM), jnp.float32),
            jax.ShapeDtypeStruct((B, 1), jnp.float32),
        ],
        scratch_shapes=[pltpu.VMEM((B, QUES_DIM), jnp.float32)],
    )(gi_all, w_hh, b_hh.reshape(1, -1), qp_w,
      qp_b.reshape(1, -1), ws_w, rw_w, rw_b.reshape(1, -1))


def _score_body(v_ref, b_ref, t_ref, wq_ref, vp_w_ref, vp_b_ref, fr_wt_ref,
                fr_b_ref, d_w_ref, sc_ref, out_ref):
    dn = (((1,), (1,)), ((), ()))
    v2 = v_ref[...].reshape(BB * K, V_DIM)
    vp = _lrelu(lax.dot_general(v2, vp_w_ref[...], dn,
                                preferred_element_type=jnp.float32)
                + vp_b_ref[...])
    vp3 = vp.reshape(BB, K, SCORE_DIM)
    ws_b = sc_ref[0, 0]
    d_b = sc_ref[0, 1]
    kappa = jnp.sum(vp3 * t_ref[...][:, None, :], axis=2) + ws_b

    ss = jnp.sum(v2 * v2, axis=1)
    inv = 1.0 / jnp.maximum(jnp.sqrt(ss), 1e-12)
    nv2 = v2 * inv[:, None]
    big = lax.dot_general(nv2, nv2, dn,
                          preferred_element_type=jnp.float32)
    vtv = jnp.concatenate(
        [big[i * K:(i + 1) * K, i * K:(i + 1) * K][None] for i in range(BB)],
        axis=0)

    bx = b_ref[...]
    x0 = bx[:, :, 0]
    y0 = bx[:, :, 1]
    x1 = bx[:, :, 2]
    y1 = bx[:, :, 3]
    area = (x1 - x0) * (y1 - y0)
    lr = jnp.minimum(x1[:, None, :], x1[:, :, None]) - \
        jnp.maximum(x0[:, None, :], x0[:, :, None])
    ud = jnp.minimum(y1[:, None, :], y1[:, :, None]) - \
        jnp.maximum(y0[:, None, :], y0[:, :, None])
    overlap = jnp.maximum(lr, 0.0) * jnp.maximum(ud, 0.0)
    a_j = area[:, None, :]
    a_i = area[:, :, None]
    iou = overlap / (a_j + a_i - overlap)
    o_ij = overlap / a_j
    o_ji = overlap / a_i

    fr_wt = fr_wt_ref[...]
    b2 = bx.reshape(BB * K, 6)
    dnn = (((1,), (0,)), ((), ()))
    u = lax.dot_general(b2, fr_wt[2:8], dnn,
                        preferred_element_type=jnp.float32)
    w2m = lax.dot_general(b2, fr_wt[8:14], dnn,
                          preferred_element_type=jnp.float32)
    u3 = u.reshape(BB, K, 100)
    w23 = w2m.reshape(BB, K, 100)

    f = lambda c: fr_wt[c].reshape(1, 1, 1, 100)
    hf = (wq_ref[...].reshape(BB, 1, 1, 1) * f(0)
          + vtv[..., None] * f(1)
          + u3[:, None, :, :]
          + w23[:, :, None, :]
          + iou[..., None] * f(14)
          + o_ij[..., None] * f(15)
          + o_ji[..., None] * f(16)
          + fr_b_ref[...].reshape(1, 1, 1, 100))
    rho = jnp.sum(_lrelu(hf) * d_w_ref[...].reshape(1, 1, 1, 100),
                  axis=3) + d_b
    out_ref[...] = jnp.concatenate([kappa[:, :, None], rho], axis=2)


def _score(v_emb, bboxes, t, wq, vp_w, vp_b, fr_w, fr_b, d_w, scalars):
    nblk = B // BB
    full = lambda s: pl.BlockSpec(s, lambda g: (0,) * len(s))
    return pl.pallas_call(
        _score_body,
        grid=(nblk,),
        in_specs=[
            pl.BlockSpec((BB, K, V_DIM), lambda g: (g, 0, 0)),
            pl.BlockSpec((BB, K, 6), lambda g: (g, 0, 0)),
            pl.BlockSpec((BB, SCORE_DIM), lambda g: (g, 0)),
            pl.BlockSpec((BB, 1), lambda g: (g, 0)),
            full((SCORE_DIM, V_DIM)),
            full((1, SCORE_DIM)),
            full((17, 100)),
            full((1, 100)),
            full((1, 100)),
            full((1, 2)),
        ],
        out_specs=pl.BlockSpec((BB, K, K + 1), lambda g: (g, 0, 0)),
        out_shape=jax.ShapeDtypeStruct((B, K, K + 1), jnp.float32),
    )(v_emb, bboxes, t, wq, vp_w, vp_b.reshape(1, -1), fr_w.T,
      fr_b.reshape(1, -1), d_w, scalars)


def kernel(v_emb, b, q, embd, w_ih, w_hh, b_ih, b_hh, vp_w, vp_b, qp_w,
           qp_b, ws_w, ws_b, rw_w, rw_b, fr_w, fr_b, d_w, d_b):
    idx = q.T.reshape(-1).astype(jnp.int32)
    embd_p = _pad_table(embd)
    xe_flat = _sc_gather(embd_p, idx)
    gi_all = _gi_all(xe_flat, w_ih, b_ih).reshape(L, B, 3 * QUES_DIM)
    t, wq = _gru(gi_all, w_hh, b_hh, qp_w, qp_b, ws_w, rw_w, rw_b)
    scalars = jnp.stack([ws_b[0], d_b[0]]).reshape(1, 2)
    return _score(v_emb, b, t, wq, vp_w, vp_b, fr_w, fr_b, d_w, scalars)
